# baseline (speedup 1.0000x reference)
import functools

import jax
import jax.numpy as jnp
from jax.experimental import pallas as pl
from jax.experimental.pallas import tpu as pltpu

_EPS = 1e-05
_TINY = float(jnp.finfo(jnp.float32).tiny)
_NEG_INF_KEY = -2139095041
_INT_MIN = -2147483648
_INT_MAX = 2147483647


def _rotl(x, r):
    return jax.lax.shift_left(x, r) | jax.lax.shift_right_logical(x, 32 - r)


def _four_rounds(x0, x1, rots):
    for r in rots:
        x0 = x0 + x1
        x1 = _rotl(x1, r)
        x1 = x1 ^ x0
    return x0, x1


def _threefry_bits_key1(j):
    ks0 = jnp.int32(0)
    ks1 = jnp.int32(1)
    ks2 = jnp.int32(0x1BD11BDA) ^ ks0 ^ ks1
    rot_a = (13, 15, 26, 6)
    rot_b = (17, 29, 16, 24)
    x0 = jnp.zeros_like(j) + ks0
    x1 = j + ks1
    x0, x1 = _four_rounds(x0, x1, rot_a)
    x0, x1 = x0 + ks1, x1 + ks2 + 1
    x0, x1 = _four_rounds(x0, x1, rot_b)
    x0, x1 = x0 + ks2, x1 + ks0 + 2
    x0, x1 = _four_rounds(x0, x1, rot_a)
    x0, x1 = x0 + ks0, x1 + ks1 + 3
    x0, x1 = _four_rounds(x0, x1, rot_b)
    x0, x1 = x0 + ks1, x1 + ks2 + 4
    x0, x1 = _four_rounds(x0, x1, rot_a)
    x0, x1 = x0 + ks2, x1 + ks0 + 5
    return x0 ^ x1


def _gumbel_from_bits(bits):
    fb = jax.lax.shift_right_logical(bits, 9) | jnp.int32(0x3F800000)
    f = jax.lax.bitcast_convert_type(fb, jnp.float32) - jnp.float32(1.0)
    u = jnp.maximum(jnp.float32(_TINY), f + jnp.float32(_TINY))
    return -jnp.log(-jnp.log(u))


def _sampler_block(logits_ref, temp_ref, topp_ref, topk_ref, out_ref, *,
                   v, rows, idx_bits):
    g = pl.program_id(0)
    x = logits_ref[...]
    temp = temp_ref[...].reshape(rows, 1)
    topp = topp_ref[...].reshape(rows, 1)
    topk = topk_ref[...].reshape(rows, 1)

    safe_t = jnp.where(temp < _EPS, jnp.float32(1.0), temp)
    lg = x / safe_t

    idx = jax.lax.broadcasted_iota(jnp.int32, (rows, v), 1)

    def greedy_fn(_):
        xm = jnp.max(x, axis=1, keepdims=True)
        return jnp.min(jnp.where(x == xm, idx, v), axis=1, keepdims=True)
    greedy = jax.lax.cond(
        jnp.any(temp < _EPS), greedy_fn,
        lambda _: jnp.zeros((rows, 1), jnp.int32), 0)

    bits = jax.lax.bitcast_convert_type(lg, jnp.int32)
    key = bits ^ (jax.lax.shift_right_arithmetic(bits, 31)
                  & jnp.int32(0x7FFFFFFF))

    step = 768
    parts = [jnp.max(key[:, s0:min(v, s0 + step)], axis=1, keepdims=True)
             for s0 in range(0, v, step)]
    smax = jnp.concatenate(parts, axis=1)
    m0key = jnp.max(smax, axis=1, keepdims=True)
    m0bits = m0key ^ (jax.lax.shift_right_arithmetic(m0key, 31)
                      & jnp.int32(0x7FFFFFFF))
    m0 = jax.lax.bitcast_convert_type(m0bits, jnp.float32)
    if len(parts) >= 64:
        def tau_body(i, acc):
            cand = acc + jax.lax.shift_left(jnp.int32(1), 31 - i)
            cnt = jnp.sum(jnp.where(smax >= cand, jnp.int32(1),
                                    jnp.int32(0)), axis=1, keepdims=True)
            return jnp.where(cnt >= 64, cand, acc)
        tau = jax.lax.fori_loop(
            0, 32, tau_body, jnp.full((rows, 1), _INT_MIN, jnp.int32))
    else:
        tau = jnp.full((rows, 1), _INT_MIN, jnp.int32)

    def _mid(a, b):
        return (a >> 1) + (b >> 1) + (a & b & jnp.int32(1))

    def _quad_cond(state):
        lo, hi = state
        return jnp.any(hi > lo + 1)

    def _quad_search(pred3, lo, hi):
        def body(state):
            lo, hi = state
            m2 = _mid(lo, hi)
            m1 = _mid(lo, m2)
            m3 = _mid(m2, hi)
            p1, p2, p3 = pred3(m1, m2, m3)
            a1 = p1
            a2 = p1 & p2
            a3 = a2 & p3
            lo2 = jnp.where(a3, m3, jnp.where(a2, m2,
                                              jnp.where(a1, m1, lo)))
            hi2 = jnp.where(~a1, m1, jnp.where(~a2, m2,
                                               jnp.where(~a3, m3, hi)))
            return lo2, hi2
        lo_f, _ = jax.lax.while_loop(_quad_cond, body, (lo, hi))
        return lo_f

    kk = jnp.maximum(topk, 1)

    def tk_pred3(m1, m2, m3):
        c1 = jnp.sum(jnp.where(key >= m1, jnp.int32(1), jnp.int32(0)),
                     axis=1, keepdims=True)
        c2 = jnp.sum(jnp.where(key >= m2, jnp.int32(1), jnp.int32(0)),
                     axis=1, keepdims=True)
        c3 = jnp.sum(jnp.where(key >= m3, jnp.int32(1), jnp.int32(0)),
                     axis=1, keepdims=True)
        return c1 >= kk, c2 >= kk, c3 >= kk
    kth_key = _quad_search(tk_pred3, tau, m0key + 1)
    keep_k = (topk == 0) | (key >= kth_key)

    ekey = jnp.where(keep_k, key, jnp.int32(_NEG_INF_KEY))
    e = jnp.exp(lg - m0)
    zsum = jnp.sum(jnp.where(keep_k, e, jnp.float32(0.0)),
                   axis=1, keepdims=True)
    ptz = topp * zsum
    lo_p = jnp.where(topk > 0, kth_key - 1, jnp.int32(_NEG_INF_KEY))

    def tp_pred3(m1, m2, m3):
        s1 = jnp.sum(jnp.where(ekey > m1, e, jnp.float32(0.0)),
                     axis=1, keepdims=True)
        s2 = jnp.sum(jnp.where(ekey > m2, e, jnp.float32(0.0)),
                     axis=1, keepdims=True)
        s3 = jnp.sum(jnp.where(ekey > m3, e, jnp.float32(0.0)),
                     axis=1, keepdims=True)
        return s1 > ptz, s2 > ptz, s3 > ptz
    below = _quad_search(tp_pred3, lo_p, m0key + 1)
    theta = below + 1
    tstar = jnp.min(jnp.where(ekey >= theta, ekey, jnp.int32(_INT_MAX)),
                    axis=1, keepdims=True)

    is_tie = ekey == tstar
    nstar = jnp.sum(jnp.where(is_tie, jnp.int32(1), jnp.int32(0)),
                    axis=1, keepdims=True)
    gsum = jnp.sum(jnp.where(ekey > tstar, e, jnp.float32(0.0)),
                   axis=1, keepdims=True)
    estar = jnp.max(jnp.where(is_tie, e, jnp.float32(0.0)),
                    axis=1, keepdims=True)
    ratio = (ptz - gsum) / estar
    cf = jnp.floor(ratio) + jnp.float32(1.0)
    nf = nstar.astype(jnp.float32)
    cf = jnp.where(estar > 0.0, jnp.minimum(cf, nf), nf)
    c = cf.astype(jnp.int32)

    def ti_search(_):
        def ti_body(i, acc):
            cand = acc + jax.lax.shift_left(jnp.int32(1), idx_bits - 1 - i)
            cnt = jnp.sum(jnp.where(is_tie & (idx >= cand), jnp.int32(1),
                                    jnp.int32(0)), axis=1, keepdims=True)
            return jnp.where(cnt >= c, cand, acc)
        return jax.lax.fori_loop(
            0, idx_bits, ti_body, jnp.zeros((rows, 1), jnp.int32))
    idx_cut = jax.lax.cond(
        jnp.any(c < nstar), ti_search,
        lambda _: jnp.zeros((rows, 1), jnp.int32), 0)

    rowbase = (g * rows
               + jax.lax.broadcasted_iota(jnp.int32, (rows, 1), 0)) * v
    bitsg = _threefry_bits_key1(rowbase + idx)
    z = _gumbel_from_bits(bitsg) + lg
    kept = (ekey > tstar) | (is_tie & (idx >= idx_cut))
    zm = jnp.max(jnp.where(kept, z, -jnp.inf), axis=1, keepdims=True)
    winner = jnp.min(jnp.where(kept & (z == zm), idx, v),
                     axis=1, keepdims=True)

    tok = jnp.where(temp < _EPS, greedy, winner)
    out_ref[...] = tok.reshape(1, rows, 1)


@jax.jit
def kernel(logits, temperature, top_p, top_k):
    logits = logits.astype(jnp.float32)
    b, v = logits.shape
    rows = 16 if b % 16 == 0 else 8
    nb = b // rows
    idx_bits = max(1, (v - 1).bit_length())
    temp3 = temperature.astype(jnp.float32).reshape(nb, rows, 1)
    topp3 = top_p.astype(jnp.float32).reshape(nb, rows, 1)
    topk3 = top_k.astype(jnp.int32).reshape(nb, rows, 1)
    out = pl.pallas_call(
        functools.partial(_sampler_block, v=v, rows=rows, idx_bits=idx_bits),
        grid=(nb,),
        in_specs=[
            pl.BlockSpec((rows, v), lambda g: (g, 0)),
            pl.BlockSpec((1, rows, 1), lambda g: (g, 0, 0)),
            pl.BlockSpec((1, rows, 1), lambda g: (g, 0, 0)),
            pl.BlockSpec((1, rows, 1), lambda g: (g, 0, 0)),
        ],
        out_specs=pl.BlockSpec((1, rows, 1), lambda g: (g, 0, 0)),
        out_shape=jax.ShapeDtypeStruct((nb, rows, 1), jnp.int32),
        compiler_params=pltpu.CompilerParams(
            vmem_limit_bytes=128 * 1024 * 1024),
    )(logits, temp3, topp3, topk3)
    return out.reshape(b)


# speedup vs baseline: 1.0183x; 1.0183x over previous
import functools

import jax
import jax.numpy as jnp
from jax.experimental import pallas as pl
from jax.experimental.pallas import tpu as pltpu

_EPS = 1e-05
_TINY = float(jnp.finfo(jnp.float32).tiny)
_NEG_INF_KEY = -2139095041
_INT_MIN = -2147483648
_INT_MAX = 2147483647


def _rotl(x, r):
    return jax.lax.shift_left(x, r) | jax.lax.shift_right_logical(x, 32 - r)


def _four_rounds(x0, x1, rots):
    for r in rots:
        x0 = x0 + x1
        x1 = _rotl(x1, r)
        x1 = x1 ^ x0
    return x0, x1


def _threefry_bits_key1(j):
    ks0 = jnp.int32(0)
    ks1 = jnp.int32(1)
    ks2 = jnp.int32(0x1BD11BDA) ^ ks0 ^ ks1
    rot_a = (13, 15, 26, 6)
    rot_b = (17, 29, 16, 24)
    x0 = jnp.zeros_like(j) + ks0
    x1 = j + ks1
    x0, x1 = _four_rounds(x0, x1, rot_a)
    x0, x1 = x0 + ks1, x1 + ks2 + 1
    x0, x1 = _four_rounds(x0, x1, rot_b)
    x0, x1 = x0 + ks2, x1 + ks0 + 2
    x0, x1 = _four_rounds(x0, x1, rot_a)
    x0, x1 = x0 + ks0, x1 + ks1 + 3
    x0, x1 = _four_rounds(x0, x1, rot_b)
    x0, x1 = x0 + ks1, x1 + ks2 + 4
    x0, x1 = _four_rounds(x0, x1, rot_a)
    x0, x1 = x0 + ks2, x1 + ks0 + 5
    return x0 ^ x1


def _gumbel_from_bits(bits):
    fb = jax.lax.shift_right_logical(bits, 9) | jnp.int32(0x3F800000)
    f = jax.lax.bitcast_convert_type(fb, jnp.float32) - jnp.float32(1.0)
    u = jnp.maximum(jnp.float32(_TINY), f + jnp.float32(_TINY))
    return -jnp.log(-jnp.log(u))


def _sampler_block(logits_ref, temp_ref, topp_ref, topk_ref, out_ref, *,
                   v, rows, idx_bits):
    g = pl.program_id(0)
    x = logits_ref[...]
    temp = temp_ref[...].reshape(rows, 1)
    topp = topp_ref[...].reshape(rows, 1)
    topk = topk_ref[...].reshape(rows, 1)

    safe_t = jnp.where(temp < _EPS, jnp.float32(1.0), temp)
    lg = x / safe_t

    idx = jax.lax.broadcasted_iota(jnp.int32, (rows, v), 1)

    def greedy_fn(_):
        xm = jnp.max(x, axis=1, keepdims=True)
        return jnp.min(jnp.where(x == xm, idx, v), axis=1, keepdims=True)
    greedy = jax.lax.cond(
        jnp.any(temp < _EPS), greedy_fn,
        lambda _: jnp.zeros((rows, 1), jnp.int32), 0)

    bits = jax.lax.bitcast_convert_type(lg, jnp.int32)
    key = bits ^ (jax.lax.shift_right_arithmetic(bits, 31)
                  & jnp.int32(0x7FFFFFFF))

    step = 768
    parts = [jnp.max(key[:, s0:min(v, s0 + step)], axis=1, keepdims=True)
             for s0 in range(0, v, step)]
    smax = jnp.concatenate(parts, axis=1)
    m0key = jnp.max(smax, axis=1, keepdims=True)
    m0bits = m0key ^ (jax.lax.shift_right_arithmetic(m0key, 31)
                      & jnp.int32(0x7FFFFFFF))
    m0 = jax.lax.bitcast_convert_type(m0bits, jnp.float32)
    if len(parts) >= 64:
        def tau_body(i, acc):
            cand = acc + jax.lax.shift_left(jnp.int32(1), 31 - i)
            cnt = jnp.sum(jnp.where(smax >= cand, jnp.int32(1),
                                    jnp.int32(0)), axis=1, keepdims=True)
            return jnp.where(cnt >= 64, cand, acc)
        tau = jax.lax.fori_loop(
            0, 32, tau_body, jnp.full((rows, 1), _INT_MIN, jnp.int32))
    else:
        tau = jnp.full((rows, 1), _INT_MIN, jnp.int32)

    kk = jnp.maximum(topk, 1)

    def tk_cond(state):
        lo, hi = state
        return jnp.any(hi > lo + 1)

    def tk_body(state):
        lo, hi = state
        mid = (lo >> 1) + (hi >> 1) + (lo & hi & jnp.int32(1))
        cnt = jnp.sum(jnp.where(key >= mid, jnp.int32(1), jnp.int32(0)),
                      axis=1, keepdims=True)
        ok = cnt >= kk
        return jnp.where(ok, mid, lo), jnp.where(ok, hi, mid)
    kth_key, _ = jax.lax.while_loop(tk_cond, tk_body, (tau, m0key + 1))
    keep_k = (topk == 0) | (key >= kth_key)

    ekey = jnp.where(keep_k, key, jnp.int32(_NEG_INF_KEY))
    e = jnp.exp(lg - m0)
    zsum = jnp.sum(jnp.where(keep_k, e, jnp.float32(0.0)),
                   axis=1, keepdims=True)
    ptz = topp * zsum
    lo_p = jnp.where(topk > 0, kth_key - 1, jnp.int32(_NEG_INF_KEY))

    def tp_cond(state):
        lo, hi = state
        return jnp.any(hi > lo + 1)

    def tp_body(state):
        lo, hi = state
        mid = (lo >> 1) + (hi >> 1) + (lo & hi & jnp.int32(1))
        s = jnp.sum(jnp.where(ekey > mid, e, jnp.float32(0.0)),
                    axis=1, keepdims=True)
        ok = s > ptz
        return jnp.where(ok, mid, lo), jnp.where(ok, hi, mid)
    below, _ = jax.lax.while_loop(
        tp_cond, tp_body, (lo_p, m0key + 1))
    theta = below + 1
    tstar = jnp.min(jnp.where(ekey >= theta, ekey, jnp.int32(_INT_MAX)),
                    axis=1, keepdims=True)

    is_tie = ekey == tstar
    nstar = jnp.sum(jnp.where(is_tie, jnp.int32(1), jnp.int32(0)),
                    axis=1, keepdims=True)
    gsum = jnp.sum(jnp.where(ekey > tstar, e, jnp.float32(0.0)),
                   axis=1, keepdims=True)
    estar = jnp.max(jnp.where(is_tie, e, jnp.float32(0.0)),
                    axis=1, keepdims=True)
    ratio = (ptz - gsum) / estar
    cf = jnp.floor(ratio) + jnp.float32(1.0)
    nf = nstar.astype(jnp.float32)
    cf = jnp.where(estar > 0.0, jnp.minimum(cf, nf), nf)
    c = cf.astype(jnp.int32)

    def ti_search(_):
        def ti_body(i, acc):
            cand = acc + jax.lax.shift_left(jnp.int32(1), idx_bits - 1 - i)
            cnt = jnp.sum(jnp.where(is_tie & (idx >= cand), jnp.int32(1),
                                    jnp.int32(0)), axis=1, keepdims=True)
            return jnp.where(cnt >= c, cand, acc)
        return jax.lax.fori_loop(
            0, idx_bits, ti_body, jnp.zeros((rows, 1), jnp.int32))
    idx_cut = jax.lax.cond(
        jnp.any(c < nstar), ti_search,
        lambda _: jnp.zeros((rows, 1), jnp.int32), 0)

    rowbase = (g * rows
               + jax.lax.broadcasted_iota(jnp.int32, (rows, 1), 0)) * v
    bitsg = _threefry_bits_key1(rowbase + idx)
    z = _gumbel_from_bits(bitsg) + lg
    kept = (ekey > tstar) | (is_tie & (idx >= idx_cut))
    zm = jnp.max(jnp.where(kept, z, -jnp.inf), axis=1, keepdims=True)
    winner = jnp.min(jnp.where(kept & (z == zm), idx, v),
                     axis=1, keepdims=True)

    tok = jnp.where(temp < _EPS, greedy, winner)
    out_ref[...] = tok.reshape(1, rows, 1)


@jax.jit
def kernel(logits, temperature, top_p, top_k):
    logits = logits.astype(jnp.float32)
    b, v = logits.shape
    rows = 16 if b % 16 == 0 else 8
    nb = b // rows
    idx_bits = max(1, (v - 1).bit_length())
    temp3 = temperature.astype(jnp.float32).reshape(nb, rows, 1)
    topp3 = top_p.astype(jnp.float32).reshape(nb, rows, 1)
    topk3 = top_k.astype(jnp.int32).reshape(nb, rows, 1)
    out = pl.pallas_call(
        functools.partial(_sampler_block, v=v, rows=rows, idx_bits=idx_bits),
        grid=(nb,),
        in_specs=[
            pl.BlockSpec((rows, v), lambda g: (g, 0)),
            pl.BlockSpec((1, rows, 1), lambda g: (g, 0, 0)),
            pl.BlockSpec((1, rows, 1), lambda g: (g, 0, 0)),
            pl.BlockSpec((1, rows, 1), lambda g: (g, 0, 0)),
        ],
        out_specs=pl.BlockSpec((1, rows, 1), lambda g: (g, 0, 0)),
        out_shape=jax.ShapeDtypeStruct((nb, rows, 1), jnp.int32),
        compiler_params=pltpu.CompilerParams(
            vmem_limit_bytes=128 * 1024 * 1024),
    )(logits, temp3, topp3, topk3)
    return out.reshape(b)
